# baseline (speedup 1.0000x reference)
import functools

import jax
import jax.numpy as jnp
from jax import lax
from jax.experimental import pallas as pl
from jax.experimental.pallas import tpu as pltpu
from jax.experimental.pallas import tpu_sc as plsc

D = 64
DP = 2 * D
NE = 1000000
B = 16384
NC = 2
NS = 16
NW = NC * NS
BW = B // NW
BH = BW // 2
CH = 128
NCH = BH // CH
L = 16

TCN = 1024


def _pack_body(x_ref, o_ref):
    t = jnp.transpose(x_ref[...], (1, 0)).reshape(TCN // 2, 2, D)
    o_ref[...] = jnp.concatenate([t[:, 0, :], t[:, 1, :]], axis=1)


def _tc_pack(tt):
    grid = (NE + TCN - 1) // TCN
    return pl.pallas_call(
        _pack_body,
        grid=(grid,),
        in_specs=[pl.BlockSpec((D, TCN), lambda i: (0, i))],
        out_specs=pl.BlockSpec((TCN // 2, DP), lambda i: (i, 0)),
        out_shape=jax.ShapeDtypeStruct((NE // 2, DP), jnp.float32),
    )(tt)


def _sc_body(hr_hbm, ho_hbm, r_hbm, tr_hbm, to_hbm, ent_hbm, rel_hbm,
             out_hbm, hi_v, ri_v, ti_v, hoff_v, toff_v, hrows, rrows, trows,
             out_v, sem):
    wid = lax.axis_index("s") * NC + lax.axis_index("c")
    iota16 = lax.iota(jnp.int32, L)

    for half in range(2):
        base = wid * BW + half * BH

        for j in range(NCH):
            pltpu.sync_copy(hr_hbm.at[pl.ds(base + j * CH, CH)], hi_v.at[j])
            pltpu.sync_copy(r_hbm.at[pl.ds(base + j * CH, CH)], ri_v.at[j])
            pltpu.sync_copy(tr_hbm.at[pl.ds(base + j * CH, CH)], ti_v.at[j])
        pltpu.sync_copy(ho_hbm.at[pl.ds(base, BH)], hoff_v)
        pltpu.sync_copy(to_hbm.at[pl.ds(base, BH)], toff_v)

        copies = []
        for j in range(NCH):
            copies.append(pltpu.async_copy(
                ent_hbm.at[hi_v.at[j]], hrows.at[pl.ds(j * CH, CH)], sem))
            copies.append(pltpu.async_copy(
                rel_hbm.at[ri_v.at[j]], rrows.at[pl.ds(j * CH, CH)], sem))
            copies.append(pltpu.async_copy(
                ent_hbm.at[ti_v.at[j]], trows.at[pl.ds(j * CH, CH)], sem))
        for c in copies:
            c.wait()

        def group(g, carry):
            res = jnp.zeros((L,), jnp.float32)
            hoff = hoff_v[pl.ds(g * L, L)]
            toff = toff_v[pl.ds(g * L, L)]
            for rr in range(L):
                i = g * L + rr
                ho = hoff[rr]
                to = toff[rr]
                acc = jnp.zeros((L,), jnp.float32)
                for c in range(D // L):
                    h = hrows[i, pl.ds(ho + c * L, L)]
                    r = rrows[i, pl.ds(c * L, L)]
                    t = trows[i, pl.ds(to + c * L, L)]
                    acc = acc + jnp.abs(h + r - t)
                s = jnp.sum(acc)
                res = jnp.where(iota16 == rr, s, res)
            out_v[pl.ds(half * BH + g * L, L)] = res
            return carry

        lax.fori_loop(0, BH // L, group, 0)

    pltpu.sync_copy(out_v, out_hbm.at[pl.ds(wid * BW, BW)])


@functools.partial(jax.jit)
def _run(head_indices, relation_indices, tail_indices, entity_table,
         relation_table):
    ent_pairs = _tc_pack(entity_table.T)
    rel_pad = jnp.pad(relation_table, ((0, 0), (0, D)))
    h_rows = head_indices // 2
    h_offs = (head_indices % 2) * D
    t_rows = tail_indices // 2
    t_offs = (tail_indices % 2) * D
    mesh = plsc.VectorSubcoreMesh(core_axis_name="c", subcore_axis_name="s")
    kfn = functools.partial(
        pl.kernel,
        mesh=mesh,
        compiler_params=pltpu.CompilerParams(
            needs_layout_passes=False,
        ),
        out_type=jax.ShapeDtypeStruct((B,), jnp.float32),
        scratch_types=[
            pltpu.VMEM((NCH, CH), jnp.int32),
            pltpu.VMEM((NCH, CH), jnp.int32),
            pltpu.VMEM((NCH, CH), jnp.int32),
            pltpu.VMEM((BH,), jnp.int32),
            pltpu.VMEM((BH,), jnp.int32),
            pltpu.VMEM((BH, DP), jnp.float32),
            pltpu.VMEM((BH, DP), jnp.float32),
            pltpu.VMEM((BH, DP), jnp.float32),
            pltpu.VMEM((BW,), jnp.float32),
            pltpu.SemaphoreType.DMA,
        ],
    )(_sc_body)
    return kfn(h_rows, h_offs, relation_indices, t_rows, t_offs, ent_pairs,
               rel_pad)


def kernel(head_indices, relation_indices, tail_indices, entity_table,
           relation_table):
    return _run(head_indices, relation_indices, tail_indices, entity_table,
                relation_table)


# speedup vs baseline: 1.2661x; 1.2661x over previous
import functools

import jax
import jax.numpy as jnp
from jax import lax
from jax.experimental import pallas as pl
from jax.experimental.pallas import tpu as pltpu
from jax.experimental.pallas import tpu_sc as plsc

D = 64
DP = 2 * D
NE = 1000000
B = 16384
NC = 2
NS = 16
NW = NC * NS
BW = B // NW
BH = BW // 2
CH = 128
NCH = BH // CH
L = 16

TCN = 1024


def _pack_body(x_ref, o_ref):
    t = jnp.transpose(x_ref[...], (1, 0)).reshape(TCN // 2, 2, D)
    o_ref[...] = jnp.concatenate([t[:, 0, :], t[:, 1, :]], axis=1)


def _tc_pack(tt):
    grid = (NE + TCN - 1) // TCN
    return pl.pallas_call(
        _pack_body,
        grid=(grid,),
        in_specs=[pl.BlockSpec((D, TCN), lambda i: (0, i))],
        out_specs=pl.BlockSpec((TCN // 2, DP), lambda i: (i, 0)),
        out_shape=jax.ShapeDtypeStruct((NE // 2, DP), jnp.float32),
    )(tt)


def _sc_body(hr_hbm, ho_hbm, r_hbm, tr_hbm, to_hbm, ent_hbm, rel_hbm,
             out_hbm, hi_v, ri_v, ti_v, hoff_v, toff_v, hrows, rrows, trows,
             out_v, sem):
    wid = lax.axis_index("s") * NC + lax.axis_index("c")
    iota16 = lax.iota(jnp.int32, L)

    for half in range(2):
        base = wid * BW + half * BH

        for j in range(NCH):
            pltpu.sync_copy(hr_hbm.at[pl.ds(base + j * CH, CH)], hi_v.at[j])
            pltpu.sync_copy(r_hbm.at[pl.ds(base + j * CH, CH)], ri_v.at[j])
            pltpu.sync_copy(tr_hbm.at[pl.ds(base + j * CH, CH)], ti_v.at[j])
        pltpu.sync_copy(ho_hbm.at[pl.ds(base, BH)], hoff_v)
        pltpu.sync_copy(to_hbm.at[pl.ds(base, BH)], toff_v)

        copies = []
        for j in range(NCH):
            copies.append(pltpu.async_copy(
                ent_hbm.at[hi_v.at[j]], hrows.at[pl.ds(j * CH, CH)], sem))
            copies.append(pltpu.async_copy(
                rel_hbm.at[ri_v.at[j]], rrows.at[pl.ds(j * CH, CH)], sem))
            copies.append(pltpu.async_copy(
                ent_hbm.at[ti_v.at[j]], trows.at[pl.ds(j * CH, CH)], sem))
        for c in copies:
            c.wait()

        def group(g, carry):
            res = jnp.zeros((L,), jnp.float32)
            hoff = hoff_v[pl.ds(g * L, L)]
            toff = toff_v[pl.ds(g * L, L)]
            for rr in range(L):
                i = g * L + rr
                ho = hoff[rr]
                to = toff[rr]
                acc = jnp.zeros((L,), jnp.float32)
                for c in range(D // L):
                    h = hrows[i, pl.ds(ho + c * L, L)]
                    r = rrows[i, pl.ds(c * L, L)]
                    t = trows[i, pl.ds(to + c * L, L)]
                    acc = acc + jnp.abs(h + r - t)
                s = jnp.sum(acc)
                res = jnp.where(iota16 == rr, s, res)
            out_v[pl.ds(half * BH + g * L, L)] = res
            return carry

        lax.fori_loop(0, BH // L, group, 0)

    pltpu.sync_copy(out_v, out_hbm.at[pl.ds(wid * BW, BW)])


@functools.partial(jax.jit)
def _run(head_indices, relation_indices, tail_indices, entity_table,
         relation_table):
    ent_pairs = entity_table.reshape(NE // 2, DP)
    rel_pad = jnp.pad(relation_table, ((0, 0), (0, D)))
    h_rows = head_indices // 2
    h_offs = (head_indices % 2) * D
    t_rows = tail_indices // 2
    t_offs = (tail_indices % 2) * D
    mesh = plsc.VectorSubcoreMesh(core_axis_name="c", subcore_axis_name="s")
    kfn = functools.partial(
        pl.kernel,
        mesh=mesh,
        compiler_params=pltpu.CompilerParams(
            needs_layout_passes=False,
        ),
        out_type=jax.ShapeDtypeStruct((B,), jnp.float32),
        scratch_types=[
            pltpu.VMEM((NCH, CH), jnp.int32),
            pltpu.VMEM((NCH, CH), jnp.int32),
            pltpu.VMEM((NCH, CH), jnp.int32),
            pltpu.VMEM((BH,), jnp.int32),
            pltpu.VMEM((BH,), jnp.int32),
            pltpu.VMEM((BH, DP), jnp.float32),
            pltpu.VMEM((BH, DP), jnp.float32),
            pltpu.VMEM((BH, DP), jnp.float32),
            pltpu.VMEM((BW,), jnp.float32),
            pltpu.SemaphoreType.DMA,
        ],
    )(_sc_body)
    return kfn(h_rows, h_offs, relation_indices, t_rows, t_offs, ent_pairs,
               rel_pad)


def kernel(head_indices, relation_indices, tail_indices, entity_table,
           relation_table):
    return _run(head_indices, relation_indices, tail_indices, entity_table,
                relation_table)


# speedup vs baseline: 1.4189x; 1.1207x over previous
import functools

import jax
import jax.numpy as jnp
from jax import lax
from jax.experimental import pallas as pl
from jax.experimental.pallas import tpu as pltpu
from jax.experimental.pallas import tpu_sc as plsc

D = 64
DP = 2 * D
B = 16384
NC = 2
NS = 16
NW = NC * NS
BW = B // NW
BH = BW // 2
CH = 128
NCH = BH // CH
L = 16


def _sc_body(h_hbm, r_hbm, t_hbm, ent_hbm, rel_hbm, out_hbm,
             hi_v, ri_v, ti_v, hrows, rrows, trows, out_v, sem):
    wid = lax.axis_index("s") * NC + lax.axis_index("c")
    iota16 = lax.iota(jnp.int32, L)

    for half in range(2):
        base = wid * BW + half * BH

        for j in range(NCH):
            pltpu.sync_copy(h_hbm.at[pl.ds(base + j * CH, CH)], hi_v.at[j])
            pltpu.sync_copy(r_hbm.at[pl.ds(base + j * CH, CH)], ri_v.at[j])
            pltpu.sync_copy(t_hbm.at[pl.ds(base + j * CH, CH)], ti_v.at[j])

        copies = []
        for j in range(NCH):
            copies.append(pltpu.async_copy(
                ent_hbm.at[hi_v.at[j]], hrows.at[pl.ds(j * CH, CH)], sem))
            copies.append(pltpu.async_copy(
                rel_hbm.at[ri_v.at[j]], rrows.at[pl.ds(j * CH, CH)], sem))
            copies.append(pltpu.async_copy(
                ent_hbm.at[ti_v.at[j]], trows.at[pl.ds(j * CH, CH)], sem))
        for c in copies:
            c.wait()

        def group(g, carry):
            res = jnp.zeros((L,), jnp.float32)
            for rr in range(L):
                i = g * L + rr
                acc = jnp.zeros((L,), jnp.float32)
                for c in range(D // L):
                    h = hrows[i, pl.ds(c * L, L)]
                    r = rrows[i, pl.ds(c * L, L)]
                    t = trows[i, pl.ds(c * L, L)]
                    acc = acc + jnp.abs(h + r - t)
                s = jnp.sum(acc)
                res = jnp.where(iota16 == rr, s, res)
            out_v[pl.ds(half * BH + g * L, L)] = res
            return carry

        lax.fori_loop(0, BH // L, group, 0)

    pltpu.sync_copy(out_v, out_hbm.at[pl.ds(wid * BW, BW)])


@functools.partial(jax.jit)
def _run(head_indices, relation_indices, tail_indices, entity_table,
         relation_table):
    ent_pad = jnp.pad(entity_table, ((0, 0), (0, D)))
    rel_pad = jnp.pad(relation_table, ((0, 0), (0, D)))
    mesh = plsc.VectorSubcoreMesh(core_axis_name="c", subcore_axis_name="s")
    kfn = functools.partial(
        pl.kernel,
        mesh=mesh,
        compiler_params=pltpu.CompilerParams(
            needs_layout_passes=False,
        ),
        out_type=jax.ShapeDtypeStruct((B,), jnp.float32),
        scratch_types=[
            pltpu.VMEM((NCH, CH), jnp.int32),
            pltpu.VMEM((NCH, CH), jnp.int32),
            pltpu.VMEM((NCH, CH), jnp.int32),
            pltpu.VMEM((BH, DP), jnp.float32),
            pltpu.VMEM((BH, DP), jnp.float32),
            pltpu.VMEM((BH, DP), jnp.float32),
            pltpu.VMEM((BW,), jnp.float32),
            pltpu.SemaphoreType.DMA,
        ],
    )(_sc_body)
    return kfn(head_indices, relation_indices, tail_indices, ent_pad,
               rel_pad)


def kernel(head_indices, relation_indices, tail_indices, entity_table,
           relation_table):
    return _run(head_indices, relation_indices, tail_indices, entity_table,
                relation_table)
